# baseline (speedup 1.0000x reference)
import functools

import jax
import jax.numpy as jnp
from jax import lax
from jax.experimental import pallas as pl
from jax.experimental.pallas import tpu as pltpu
from jax.experimental.pallas import tpu_sc as plsc

N = 32768
B = 16
GRID = 256.0
D_OUT = 128
ROWS_2D = N // 128

BLK = 8192
SUB = BLK // 128
NBLK = N // BLK

CH = 1024
NEG = -3.0e38


def _sc_body(p0_hbm, p1_hbm, p2_hbm, p3_hbm, off_hbm, m01_hbm,
             off_v, a0_v, a1_v, a2_v, a3_v, loc_v, shared, red_v):
    c = lax.axis_index("c")
    s = lax.axis_index("s")
    pltpu.sync_copy(off_hbm, off_v.at[pl.ds(0, 16)])
    io = lax.iota(jnp.int32, 16)
    svec = jnp.full((16,), s, jnp.int32)
    cvec = jnp.full((16,), c, jnp.int32)
    wc0 = jnp.where(cvec == 0, 1.0, 0.0)
    wc1 = 1.0 - wc0
    ohs = jnp.where(io == svec, 1.0, 0.0)
    hi = off_v[pl.ds(s, 16)][0]
    lo = jnp.where(s > 0, off_v[pl.ds(jnp.maximum(s - 1, 0), 16)][0], 0)
    lo_v = jnp.full((16,), lo, jnp.int32)
    hi_v = jnp.full((16,), hi, jnp.int32)
    k0 = lo // CH
    k1 = jnp.where(hi > lo, (hi + (CH - 1)) // CH, k0)

    def chunk(k, acc):
        base = k * CH
        pltpu.sync_copy(p0_hbm.at[pl.ds(base, CH)], a0_v)
        pltpu.sync_copy(p1_hbm.at[pl.ds(base, CH)], a1_v)
        pltpu.sync_copy(p2_hbm.at[pl.ds(base, CH)], a2_v)
        pltpu.sync_copy(p3_hbm.at[pl.ds(base, CH)], a3_v)
        rv0 = io + jnp.full((16,), base, jnp.int32)

        def inner(j, carry):
            acc_i, rv = carry
            u0 = a0_v[pl.ds(j * 16, 16)]
            u1 = a1_v[pl.ds(j * 16, 16)]
            u2 = a2_v[pl.ds(j * 16, 16)]
            u3 = a3_v[pl.ds(j * 16, 16)]
            sx = (u0 + u2) * wc0 + (u1 + u3) * wc1
            mask = (rv >= lo_v) & (rv < hi_v)
            acc_i = jnp.maximum(acc_i, jnp.where(mask, sx, NEG))
            return (acc_i, rv + 16)

        acc, _ = lax.fori_loop(0, CH // 16, inner, (acc, rv0))
        return acc

    acc = lax.fori_loop(k0, k1, chunk, jnp.full((16,), NEG, jnp.float32))
    for d in (1, 2, 4, 8):
        acc = jnp.maximum(acc, acc.at[io ^ d].get(mode="promise_in_bounds"))
    acc = acc * jnp.float32(0.5)
    loc_v[...] = acc * ohs + jnp.float32(NEG) * (1.0 - ohs)
    pltpu.sync_copy(loc_v, shared.at[pl.ds(s * 16, 16)])
    plsc.subcore_barrier()

    @pl.when(s == 0)
    def _combine():
        pltpu.sync_copy(shared, red_v)
        m = red_v[pl.ds(0, 16)]
        for i in range(1, 16):
            m = jnp.maximum(m, red_v[pl.ds(i * 16, 16)])
        loc_v[...] = m
        pltpu.sync_copy(loc_v, m01_hbm.at[c])


def _sc_seg_maxes(pc0, pc1, pc2, pc3, offset):
    mesh = plsc.VectorSubcoreMesh(core_axis_name="c", subcore_axis_name="s")
    f = functools.partial(
        pl.kernel,
        mesh=mesh,
        out_type=jax.ShapeDtypeStruct((2, 16), jnp.float32),
        scratch_types=[
            pltpu.VMEM((32,), jnp.int32),
            pltpu.VMEM((CH,), jnp.float32),
            pltpu.VMEM((CH,), jnp.float32),
            pltpu.VMEM((CH,), jnp.float32),
            pltpu.VMEM((CH,), jnp.float32),
            pltpu.VMEM((16,), jnp.float32),
            pltpu.VMEM_SHARED((256,), jnp.float32),
            pltpu.VMEM((256,), jnp.float32),
        ],
    )(_sc_body)
    return f(pc0, pc1, pc2, pc3, offset)


def _tc_body(off_ref, m01_ref, p0, p1, p2, p3, w_ref, b_ref, out_ref,
             xn_s, yn_s, bi_s):
    i = pl.program_id(0)

    @pl.when(i == 0)
    def _norm_cols():
        x = (p0[...] + p2[...]) * 0.5
        y = (p1[...] + p3[...]) * 0.5
        ii = lax.broadcasted_iota(jnp.int32, (ROWS_2D, 128), 0)
        jj = lax.broadcasted_iota(jnp.int32, (ROWS_2D, 128), 1)
        r = ii * 128 + jj
        seg = jnp.zeros((ROWS_2D, 128), jnp.int32)
        for k in range(B):
            seg = seg + (r >= off_ref[k]).astype(jnp.int32)
        m0v = jnp.zeros((ROWS_2D, 128), jnp.float32)
        m1v = jnp.zeros((ROWS_2D, 128), jnp.float32)
        for k in range(B):
            mask = seg == k
            m0v = jnp.where(mask, m01_ref[0, k], m0v)
            m1v = jnp.where(mask, m01_ref[1, k], m1v)
        xn_s[...] = x / m0v * GRID
        yn_s[...] = y / m1v * GRID
        bi_s[...] = seg.astype(jnp.float32)

    a0 = p0[pl.ds(i * SUB, SUB), :]
    a1 = p1[pl.ds(i * SUB, SUB), :]
    a2 = p2[pl.ds(i * SUB, SUB), :]
    a3 = p3[pl.ds(i * SUB, SUB), :]
    wdt = a2 - a0
    hgt = a3 - a1
    area = wdt * hgt
    xn = xn_s[pl.ds(i * SUB, SUB), :]
    yn = yn_s[pl.ds(i * SUB, SUB), :]
    bi = bi_s[pl.ds(i * SUB, SUB), :]
    feats = [bi, xn, yn, a0, a1, a2, a3, wdt, hgt, area]
    wmat = w_ref[...]
    bvec = b_ref[...]
    for g in range(SUB):
        ft = jnp.concatenate([f[g : g + 1, :] for f in feats], axis=0)
        o = lax.dot_general(
            ft, wmat, (((0,), (0,)), ((), ())),
            preferred_element_type=jnp.float32,
        )
        out_ref[pl.ds(g * 128, 128), :] = jnp.maximum(o + bvec, 0.0)


@jax.jit
def kernel(pc0, pc1, pc2, pc3, offset, W, b):
    m01 = _sc_seg_maxes(pc0, pc1, pc2, pc3, offset)
    pcs2d = [a.reshape(ROWS_2D, 128) for a in (pc0, pc1, pc2, pc3)]
    out = pl.pallas_call(
        _tc_body,
        grid_spec=pltpu.PrefetchScalarGridSpec(
            num_scalar_prefetch=2,
            grid=(NBLK,),
            in_specs=[pl.BlockSpec((ROWS_2D, 128), lambda i, *_: (0, 0))] * 4
            + [
                pl.BlockSpec((10, D_OUT), lambda i, *_: (0, 0)),
                pl.BlockSpec((1, D_OUT), lambda i, *_: (0, 0)),
            ],
            out_specs=pl.BlockSpec((BLK, D_OUT), lambda i, *_: (i, 0)),
            scratch_shapes=[
                pltpu.VMEM((ROWS_2D, 128), jnp.float32),
                pltpu.VMEM((ROWS_2D, 128), jnp.float32),
                pltpu.VMEM((ROWS_2D, 128), jnp.float32),
            ],
        ),
        out_shape=jax.ShapeDtypeStruct((N, D_OUT), jnp.float32),
    )(offset, m01, *pcs2d, W, b.reshape(1, D_OUT))
    return out


# speedup vs baseline: 3.3477x; 3.3477x over previous
import functools

import jax
import jax.numpy as jnp
from jax import lax
from jax.experimental import pallas as pl
from jax.experimental.pallas import tpu as pltpu

N = 32768
B = 16
GRID = 256.0
D_OUT = 128
ROWS_2D = N // 128

BLK = 8192
SUB = BLK // 128
NBLK = N // BLK


def _body(off_ref, p0, p1, p2, p3, w_ref, b_ref, out_ref, m0s, m1s, xn_s, yn_s, bi_s):
    i = pl.program_id(0)

    @pl.when(i == 0)
    def _seg_maxes():
        x = (p0[...] + p2[...]) * 0.5
        y = (p1[...] + p3[...]) * 0.5
        ii = lax.broadcasted_iota(jnp.int32, (ROWS_2D, 128), 0)
        jj = lax.broadcasted_iota(jnp.int32, (ROWS_2D, 128), 1)
        r = ii * 128 + jj
        seg = jnp.zeros((ROWS_2D, 128), jnp.int32)
        for k in range(B):
            seg = seg + (r >= off_ref[k]).astype(jnp.int32)
        neg = jnp.float32(-1e30)
        for k in range(B):
            mask = seg == k
            m0s[k] = jnp.max(jnp.where(mask, x, neg))
            m1s[k] = jnp.max(jnp.where(mask, y, neg))
        m0v = jnp.zeros((ROWS_2D, 128), jnp.float32)
        m1v = jnp.zeros((ROWS_2D, 128), jnp.float32)
        for k in range(B):
            mask = seg == k
            m0v = jnp.where(mask, m0s[k], m0v)
            m1v = jnp.where(mask, m1s[k], m1v)
        xn_s[...] = x / m0v * GRID
        yn_s[...] = y / m1v * GRID
        bi_s[...] = seg.astype(jnp.float32)

    a0 = p0[pl.ds(i * SUB, SUB), :]
    a1 = p1[pl.ds(i * SUB, SUB), :]
    a2 = p2[pl.ds(i * SUB, SUB), :]
    a3 = p3[pl.ds(i * SUB, SUB), :]
    wdt = a2 - a0
    hgt = a3 - a1
    area = wdt * hgt
    xn = xn_s[pl.ds(i * SUB, SUB), :]
    yn = yn_s[pl.ds(i * SUB, SUB), :]
    bi = bi_s[pl.ds(i * SUB, SUB), :]
    feats = [bi, xn, yn, a0, a1, a2, a3, wdt, hgt, area]
    wmat = w_ref[...]
    bvec = b_ref[...]
    for g in range(SUB):
        ft = jnp.concatenate([f[g : g + 1, :] for f in feats], axis=0)
        o = lax.dot_general(
            ft, wmat, (((0,), (0,)), ((), ())),
            preferred_element_type=jnp.float32,
        )
        out_ref[pl.ds(g * 128, 128), :] = jnp.maximum(o + bvec, 0.0)


@jax.jit
def kernel(pc0, pc1, pc2, pc3, offset, W, b):
    pcs2d = [a.reshape(ROWS_2D, 128) for a in (pc0, pc1, pc2, pc3)]
    out = pl.pallas_call(
        _body,
        grid_spec=pltpu.PrefetchScalarGridSpec(
            num_scalar_prefetch=1,
            grid=(NBLK,),
            in_specs=[pl.BlockSpec((ROWS_2D, 128), lambda i, *_: (0, 0))] * 4
            + [
                pl.BlockSpec((10, D_OUT), lambda i, *_: (0, 0)),
                pl.BlockSpec((1, D_OUT), lambda i, *_: (0, 0)),
            ],
            out_specs=pl.BlockSpec((BLK, D_OUT), lambda i, *_: (i, 0)),
            scratch_shapes=[
                pltpu.SMEM((B,), jnp.float32),
                pltpu.SMEM((B,), jnp.float32),
                pltpu.VMEM((ROWS_2D, 128), jnp.float32),
                pltpu.VMEM((ROWS_2D, 128), jnp.float32),
                pltpu.VMEM((ROWS_2D, 128), jnp.float32),
            ],
        ),
        out_shape=jax.ShapeDtypeStruct((N, D_OUT), jnp.float32),
    )(offset, *pcs2d, W, b.reshape(1, D_OUT))
    return out


# speedup vs baseline: 3.3793x; 1.0095x over previous
import jax
import jax.numpy as jnp
from jax import lax
from jax.experimental import pallas as pl
from jax.experimental.pallas import tpu as pltpu

N = 32768
B = 16
GRID = 256.0
D_OUT = 128
ROWS_2D = N // 128

BLK = 8192
SUB = BLK // 128
NBLK = N // BLK


def _body(off_ref, p0, p1, p2, p3, w_ref, b_ref, out_ref, m0s, m1s):
    i = pl.program_id(0)

    @pl.when(i == 0)
    def _seg_maxes():
        x = (p0[...] + p2[...]) * 0.5
        y = (p1[...] + p3[...]) * 0.5
        ii = lax.broadcasted_iota(jnp.int32, (ROWS_2D, 128), 0)
        jj = lax.broadcasted_iota(jnp.int32, (ROWS_2D, 128), 1)
        r = ii * 128 + jj
        seg = jnp.zeros((ROWS_2D, 128), jnp.int32)
        for k in range(B):
            seg = seg + (r >= off_ref[k]).astype(jnp.int32)
        neg = jnp.float32(-1e30)
        for k in range(B):
            mask = seg == k
            m0s[k] = jnp.max(jnp.where(mask, x, neg))
            m1s[k] = jnp.max(jnp.where(mask, y, neg))

    a0 = p0[pl.ds(i * SUB, SUB), :]
    a1 = p1[pl.ds(i * SUB, SUB), :]
    a2 = p2[pl.ds(i * SUB, SUB), :]
    a3 = p3[pl.ds(i * SUB, SUB), :]
    x = (a0 + a2) * 0.5
    y = (a1 + a3) * 0.5
    wdt = a2 - a0
    hgt = a3 - a1
    area = wdt * hgt
    ii = lax.broadcasted_iota(jnp.int32, (SUB, 128), 0)
    jj = lax.broadcasted_iota(jnp.int32, (SUB, 128), 1)
    r = i * BLK + ii * 128 + jj
    seg = jnp.zeros((SUB, 128), jnp.int32)
    for k in range(B):
        seg = seg + (r >= off_ref[k]).astype(jnp.int32)
    m0v = jnp.zeros((SUB, 128), jnp.float32)
    m1v = jnp.zeros((SUB, 128), jnp.float32)
    for k in range(B):
        mask = seg == k
        m0v = jnp.where(mask, m0s[k], m0v)
        m1v = jnp.where(mask, m1s[k], m1v)
    xn = x / m0v * GRID
    yn = y / m1v * GRID
    bi = seg.astype(jnp.float32)
    feats = [bi, xn, yn, a0, a1, a2, a3, wdt, hgt, area]
    wmat = w_ref[...]
    bvec = b_ref[...]
    for g in range(SUB):
        ft = jnp.concatenate([f[g : g + 1, :] for f in feats], axis=0)
        o = lax.dot_general(
            ft, wmat, (((0,), (0,)), ((), ())),
            preferred_element_type=jnp.float32,
        )
        out_ref[pl.ds(g * 128, 128), :] = jnp.maximum(o + bvec, 0.0)


@jax.jit
def kernel(pc0, pc1, pc2, pc3, offset, W, b):
    pcs2d = [a.reshape(ROWS_2D, 128) for a in (pc0, pc1, pc2, pc3)]
    out = pl.pallas_call(
        _body,
        grid_spec=pltpu.PrefetchScalarGridSpec(
            num_scalar_prefetch=1,
            grid=(NBLK,),
            in_specs=[pl.BlockSpec((ROWS_2D, 128), lambda i, *_: (0, 0))] * 4
            + [
                pl.BlockSpec((10, D_OUT), lambda i, *_: (0, 0)),
                pl.BlockSpec((1, D_OUT), lambda i, *_: (0, 0)),
            ],
            out_specs=pl.BlockSpec((BLK, D_OUT), lambda i, *_: (i, 0)),
            scratch_shapes=[
                pltpu.SMEM((B,), jnp.float32),
                pltpu.SMEM((B,), jnp.float32),
            ],
        ),
        out_shape=jax.ShapeDtypeStruct((N, D_OUT), jnp.float32),
    )(offset, *pcs2d, W, b.reshape(1, D_OUT))
    return out
